# baseline (speedup 1.0000x reference)
import functools

import jax
import jax.numpy as jnp
from jax import lax
from jax.experimental import pallas as pl
from jax.experimental.pallas import tpu as pltpu
from jax.experimental.pallas import tpu_sc as plsc

_NC = 2
_NS = 16
_L = 16
_NW = _NC * _NS
_CHUNK = 64


def _sc_body(K, D, t_hbm, c_hbm, n_hbm, iw_hbm, ow_hbm, pos_out, neg_out,
             t_idx, c_idx, n_idx, v_rows, up_rows, un_rows, pos_part,
             neg_part, sem):
    B = t_hbm.shape[0]
    per_w = B // _NW
    n_chunks = per_w // _CHUNK
    nd = D // _L
    ng = (_CHUNK * K) // 128
    wid = lax.axis_index("s") * _NC + lax.axis_index("c")

    def chunk(it, carry):
        base = wid * per_w + it * _CHUNK
        pltpu.sync_copy(t_hbm.at[pl.ds(base, _CHUNK)], t_idx)
        pltpu.sync_copy(c_hbm.at[pl.ds(base, _CHUNK)], c_idx)
        pltpu.sync_copy(n_hbm.at[pl.ds(base * K, _CHUNK * K)], n_idx)
        cps = [
            pltpu.async_copy(iw_hbm.at[t_idx], v_rows, sem),
            pltpu.async_copy(ow_hbm.at[c_idx], up_rows, sem),
        ]
        for g in range(ng):
            cps.append(pltpu.async_copy(
                ow_hbm.at[n_idx.at[pl.ds(g * 128, 128)]],
                un_rows.at[pl.ds(g * 128, 128)], sem))
        for cp in cps:
            cp.wait()

        def elem(b, carry2):
            v = [v_rows[b, pl.ds(j * _L, _L)] for j in range(nd)]
            u = [up_rows[b, pl.ds(j * _L, _L)] for j in range(nd)]
            acc = v[0] * u[0]
            for j in range(1, nd):
                acc = acc + v[j] * u[j]
            pos_part[pl.ds(b * _L, _L)] = acc
            for k in range(K):
                w = [un_rows[b * K + k, pl.ds(j * _L, _L)] for j in range(nd)]
                nacc = v[0] * w[0]
                for j in range(1, nd):
                    nacc = nacc + v[j] * w[j]
                neg_part[pl.ds((b * K + k) * _L, _L)] = nacc
            return carry2

        lax.fori_loop(0, _CHUNK, elem, 0)
        pltpu.sync_copy(pos_part, pos_out.at[pl.ds(base * _L, _CHUNK * _L)])
        pltpu.sync_copy(neg_part,
                        neg_out.at[pl.ds(base * K * _L, _CHUNK * K * _L)])
        return carry

    lax.fori_loop(0, n_chunks, chunk, 0)


@functools.lru_cache(maxsize=None)
def _make_sc(B, K, D):
    mesh = plsc.VectorSubcoreMesh(core_axis_name="c", subcore_axis_name="s")
    return pl.kernel(
        functools.partial(_sc_body, K, D),
        out_type=[
            jax.ShapeDtypeStruct((B * _L,), jnp.float32),
            jax.ShapeDtypeStruct((B * K * _L,), jnp.float32),
        ],
        mesh=mesh,
        compiler_params=pltpu.CompilerParams(use_tc_tiling_on_sc=False),
        scratch_types=[
            pltpu.VMEM((_CHUNK,), jnp.int32),
            pltpu.VMEM((_CHUNK,), jnp.int32),
            pltpu.VMEM((_CHUNK * K,), jnp.int32),
            pltpu.VMEM((_CHUNK, D), jnp.float32),
            pltpu.VMEM((_CHUNK, D), jnp.float32),
            pltpu.VMEM((_CHUNK * K, D), jnp.float32),
            pltpu.VMEM((_CHUNK * _L,), jnp.float32),
            pltpu.VMEM((_CHUNK * K * _L,), jnp.float32),
            pltpu.SemaphoreType.DMA,
        ],
    )


def _tr_body(a_ref, b_ref, ao_ref, bo_ref):
    d = a_ref.shape[0]
    eye = (lax.broadcasted_iota(jnp.int32, (d, d), 0)
           == lax.broadcasted_iota(jnp.int32, (d, d), 1)).astype(jnp.float32)
    dn = (((0,), (0,)), ((), ()))
    ao_ref[:] = lax.dot_general(a_ref[:], eye, dn,
                                preferred_element_type=jnp.float32)
    bo_ref[:] = lax.dot_general(b_ref[:], eye, dn,
                                preferred_element_type=jnp.float32)


def _tc_transpose(iw_t, ow_t):
    D, V = iw_t.shape
    blk = 8192
    grid = (V + blk - 1) // blk
    return pl.pallas_call(
        _tr_body,
        grid=(grid,),
        in_specs=[
            pl.BlockSpec((D, blk), lambda i: (0, i)),
            pl.BlockSpec((D, blk), lambda i: (0, i)),
        ],
        out_specs=[
            pl.BlockSpec((blk, D), lambda i: (i, 0)),
            pl.BlockSpec((blk, D), lambda i: (i, 0)),
        ],
        out_shape=[
            jax.ShapeDtypeStruct((V, D), jnp.float32),
            jax.ShapeDtypeStruct((V, D), jnp.float32),
        ],
    )(iw_t, ow_t)


def _loss_body(B, pos_ref, neg_ref, out_ref):
    i = pl.program_id(0)

    @pl.when(i == 0)
    def _init():
        out_ref[0, 0] = 0.0

    col = lax.broadcasted_iota(jnp.int32, (128, 128), 0) // _L
    row = lax.broadcasted_iota(jnp.int32, (128, 128), 1)
    sel = jnp.where((col == row) & (row < 128 // _L), 1.0, 0.0)
    valid = lax.broadcasted_iota(jnp.int32, (1, 128), 1) < 128 // _L

    def logsig(x):
        return jnp.minimum(x, 0.0) - jnp.log1p(jnp.exp(-jnp.abs(x)))

    ps = jnp.dot(pos_ref[:], sel, preferred_element_type=jnp.float32)
    ns = jnp.dot(neg_ref[:], sel, preferred_element_type=jnp.float32)
    tot = (jnp.sum(jnp.where(valid, logsig(ps), 0.0))
           + jnp.sum(jnp.where(valid, logsig(-ns), 0.0)))
    out_ref[0, 0] += -tot / B


def _tc_loss(pos2d, neg2d, B):
    n_steps = 8
    pb = pos2d.shape[0] // n_steps
    nb = neg2d.shape[0] // n_steps
    out = pl.pallas_call(
        functools.partial(_loss_body, B),
        grid=(n_steps,),
        in_specs=[
            pl.BlockSpec((pb, 128), lambda i: (i, 0)),
            pl.BlockSpec((nb, 128), lambda i: (i, 0)),
        ],
        out_specs=pl.BlockSpec(memory_space=pltpu.SMEM),
        out_shape=jax.ShapeDtypeStruct((1, 1), jnp.float32),
    )(pos2d, neg2d)
    return out[0, 0]


def kernel(targets, contexts, negatives, input_w, output_w):
    B = targets.shape[0]
    K = negatives.shape[1]
    D = input_w.shape[1]
    neg_flat = negatives.reshape(B * K)
    iw_rm, ow_rm = _tc_transpose(input_w.T, output_w.T)
    pos_part, neg_part = _make_sc(B, K, D)(
        targets, contexts, neg_flat, iw_rm, ow_rm)
    pos2d = pos_part.reshape(-1, 128)
    neg2d = neg_part.reshape(-1, 128)
    return _tc_loss(pos2d, neg2d, B)


# speedup vs baseline: 2.2043x; 2.2043x over previous
import functools

import jax
import jax.numpy as jnp
from jax import lax
from jax.experimental import pallas as pl
from jax.experimental.pallas import tpu as pltpu
from jax.experimental.pallas import tpu_sc as plsc

_NC = 2
_NS = 16
_L = 16
_NW = _NC * _NS
_CHUNK = 64


def _sc_body(K, D, H, t_hbm, c_hbm, n_hbm, iw_hbm, ow_hbm, pos_out, neg_out,
             t_idx, c_idx, n_idx, tq, cq, nq, ht, hc, hn, v_rows, up_rows,
             un_rows, pos_part, neg_part, sem):
    B = t_hbm.shape[0]
    per_w = B // _NW
    n_chunks = per_w // _CHUNK
    nd = D // _L
    ng = (_CHUNK * K) // 128
    wid = lax.axis_index("s") * _NC + lax.axis_index("c")

    def split_idx(idx_ref, q_ref, h_ref, n16):
        for i in range(n16):
            x = idx_ref[pl.ds(i * _L, _L)]
            m = x >= H
            q_ref[pl.ds(i * _L, _L)] = jnp.where(m, x - H, x)
            h_ref[pl.ds(i * _L, _L)] = jnp.where(m, 1.0, 0.0)

    def chunk(it, carry):
        base = wid * per_w + it * _CHUNK
        pltpu.sync_copy(t_hbm.at[pl.ds(base, _CHUNK)], t_idx)
        pltpu.sync_copy(c_hbm.at[pl.ds(base, _CHUNK)], c_idx)
        pltpu.sync_copy(n_hbm.at[pl.ds(base * K, _CHUNK * K)], n_idx)
        split_idx(t_idx, tq, ht, _CHUNK // _L)
        split_idx(c_idx, cq, hc, _CHUNK // _L)
        split_idx(n_idx, nq, hn, (_CHUNK * K) // _L)
        cps = [
            pltpu.async_copy(iw_hbm.at[tq], v_rows, sem),
            pltpu.async_copy(ow_hbm.at[cq], up_rows, sem),
        ]
        for g in range(ng):
            cps.append(pltpu.async_copy(
                ow_hbm.at[nq.at[pl.ds(g * 128, 128)]],
                un_rows.at[pl.ds(g * 128, 128)], sem))
        for cp in cps:
            cp.wait()

        def splat(h_ref, i):
            grp = (i // _L) * _L
            hvec = h_ref[pl.ds(grp, _L)]
            return hvec.at[jnp.broadcast_to(i - grp, (_L,))].get(
                mode="promise_in_bounds")

        def blend(rows, b, h_s):
            out = []
            for j in range(nd):
                lo = rows[b, pl.ds(j * _L, _L)]
                hi = rows[b, pl.ds(D + j * _L, _L)]
                out.append(lo + h_s * (hi - lo))
            return out

        def elem(b, carry2):
            v = blend(v_rows, b, splat(ht, b))
            u = blend(up_rows, b, splat(hc, b))
            acc = v[0] * u[0]
            for j in range(1, nd):
                acc = acc + v[j] * u[j]
            pos_part[pl.ds(b * _L, _L)] = acc
            for k in range(K):
                r = b * K + k
                w = blend(un_rows, r, splat(hn, r))
                nacc = v[0] * w[0]
                for j in range(1, nd):
                    nacc = nacc + v[j] * w[j]
                neg_part[pl.ds(r * _L, _L)] = nacc
            return carry2

        lax.fori_loop(0, _CHUNK, elem, 0)
        pltpu.sync_copy(pos_part, pos_out.at[pl.ds(base * _L, _CHUNK * _L)])
        pltpu.sync_copy(neg_part,
                        neg_out.at[pl.ds(base * K * _L, _CHUNK * K * _L)])
        return carry

    lax.fori_loop(0, n_chunks, chunk, 0)


@functools.lru_cache(maxsize=None)
def _make_sc(B, K, D, H):
    mesh = plsc.VectorSubcoreMesh(core_axis_name="c", subcore_axis_name="s")
    return pl.kernel(
        functools.partial(_sc_body, K, D, H),
        out_type=[
            jax.ShapeDtypeStruct((B * _L,), jnp.float32),
            jax.ShapeDtypeStruct((B * K * _L,), jnp.float32),
        ],
        mesh=mesh,
        compiler_params=pltpu.CompilerParams(use_tc_tiling_on_sc=False),
        scratch_types=[
            pltpu.VMEM((_CHUNK,), jnp.int32),
            pltpu.VMEM((_CHUNK,), jnp.int32),
            pltpu.VMEM((_CHUNK * K,), jnp.int32),
            pltpu.VMEM((_CHUNK,), jnp.int32),
            pltpu.VMEM((_CHUNK,), jnp.int32),
            pltpu.VMEM((_CHUNK * K,), jnp.int32),
            pltpu.VMEM((_CHUNK,), jnp.float32),
            pltpu.VMEM((_CHUNK,), jnp.float32),
            pltpu.VMEM((_CHUNK * K,), jnp.float32),
            pltpu.VMEM((_CHUNK, 2 * D), jnp.float32),
            pltpu.VMEM((_CHUNK, 2 * D), jnp.float32),
            pltpu.VMEM((_CHUNK * K, 2 * D), jnp.float32),
            pltpu.VMEM((_CHUNK * _L,), jnp.float32),
            pltpu.VMEM((_CHUNK * K * _L,), jnp.float32),
            pltpu.SemaphoreType.DMA,
        ],
    )


def _tr_body(alo_ref, ahi_ref, blo_ref, bhi_ref, ao_ref, bo_ref):
    d = alo_ref.shape[0]
    eye = (lax.broadcasted_iota(jnp.int32, (d, d), 0)
           == lax.broadcasted_iota(jnp.int32, (d, d), 1)).astype(jnp.float32)
    dn = (((0,), (0,)), ((), ()))

    def tr(x):
        return lax.dot_general(x, eye, dn, preferred_element_type=jnp.float32)

    ao_ref[:] = jnp.concatenate([tr(alo_ref[:]), tr(ahi_ref[:])], axis=1)
    bo_ref[:] = jnp.concatenate([tr(blo_ref[:]), tr(bhi_ref[:])], axis=1)


_TBLK = 4096


def _pair_offset(V):
    return (V // (2 * _TBLK)) * _TBLK


def _tc_transpose(iw_t, ow_t):
    D, V = iw_t.shape
    hp_b = _pair_offset(V) // _TBLK
    R = V - hp_b * _TBLK
    grid = (R + _TBLK - 1) // _TBLK
    return pl.pallas_call(
        _tr_body,
        grid=(grid,),
        in_specs=[
            pl.BlockSpec((D, _TBLK), lambda i: (0, i)),
            pl.BlockSpec((D, _TBLK), lambda i, o=hp_b: (0, o + i)),
            pl.BlockSpec((D, _TBLK), lambda i: (0, i)),
            pl.BlockSpec((D, _TBLK), lambda i, o=hp_b: (0, o + i)),
        ],
        out_specs=[
            pl.BlockSpec((_TBLK, 2 * D), lambda i: (i, 0)),
            pl.BlockSpec((_TBLK, 2 * D), lambda i: (i, 0)),
        ],
        out_shape=[
            jax.ShapeDtypeStruct((R, 2 * D), jnp.float32),
            jax.ShapeDtypeStruct((R, 2 * D), jnp.float32),
        ],
    )(iw_t, iw_t, ow_t, ow_t)


def _loss_body(B, pos_ref, neg_ref, out_ref):
    i = pl.program_id(0)

    @pl.when(i == 0)
    def _init():
        out_ref[0, 0] = 0.0

    col = lax.broadcasted_iota(jnp.int32, (128, 128), 0) // _L
    row = lax.broadcasted_iota(jnp.int32, (128, 128), 1)
    sel = jnp.where((col == row) & (row < 128 // _L), 1.0, 0.0)
    valid = lax.broadcasted_iota(jnp.int32, (1, 128), 1) < 128 // _L

    def logsig(x):
        return jnp.minimum(x, 0.0) - jnp.log1p(jnp.exp(-jnp.abs(x)))

    ps = jnp.dot(pos_ref[:], sel, preferred_element_type=jnp.float32)
    ns = jnp.dot(neg_ref[:], sel, preferred_element_type=jnp.float32)
    tot = (jnp.sum(jnp.where(valid, logsig(ps), 0.0))
           + jnp.sum(jnp.where(valid, logsig(-ns), 0.0)))
    out_ref[0, 0] += -tot / B


def _tc_loss(pos2d, neg2d, B):
    n_steps = 8
    pb = pos2d.shape[0] // n_steps
    nb = neg2d.shape[0] // n_steps
    out = pl.pallas_call(
        functools.partial(_loss_body, B),
        grid=(n_steps,),
        in_specs=[
            pl.BlockSpec((pb, 128), lambda i: (i, 0)),
            pl.BlockSpec((nb, 128), lambda i: (i, 0)),
        ],
        out_specs=pl.BlockSpec(memory_space=pltpu.SMEM),
        out_shape=jax.ShapeDtypeStruct((1, 1), jnp.float32),
    )(pos2d, neg2d)
    return out[0, 0]


def kernel(targets, contexts, negatives, input_w, output_w):
    B = targets.shape[0]
    K = negatives.shape[1]
    D = input_w.shape[1]
    neg_flat = negatives.reshape(B * K)
    iw_pk, ow_pk = _tc_transpose(input_w.T, output_w.T)
    pos_part, neg_part = _make_sc(B, K, D, _pair_offset(input_w.shape[0]))(
        targets, contexts, neg_flat, iw_pk, ow_pk)
    pos2d = pos_part.reshape(-1, 128)
    neg2d = neg_part.reshape(-1, 128)
    return _tc_loss(pos2d, neg2d, B)


# speedup vs baseline: 2.3161x; 1.0507x over previous
import functools

import jax
import jax.numpy as jnp
from jax import lax
from jax.experimental import pallas as pl
from jax.experimental.pallas import tpu as pltpu
from jax.experimental.pallas import tpu_sc as plsc

_NC = 2
_NS = 16
_L = 16
_NW = _NC * _NS
_CHUNK = 64


def _sc_body(K, D, H, t_hbm, c_hbm, n_hbm, iw_hbm, ow_hbm, pos_out, neg_out,
             t_idx, c_idx, n_idx, tq, cq, nq, ht, hc, hn, v_rows, up_rows,
             un_rows, pos_part, neg_part, sem):
    B = t_hbm.shape[0]
    per_w = B // _NW
    n_chunks = per_w // _CHUNK
    nd = D // _L
    ng = (_CHUNK * K) // 128
    wid = lax.axis_index("s") * _NC + lax.axis_index("c")

    def split_idx(idx_ref, q_ref, h_ref, n16):
        for i in range(n16):
            x = idx_ref[pl.ds(i * _L, _L)]
            m = x >= H
            q_ref[pl.ds(i * _L, _L)] = jnp.where(m, x - H, x)
            h_ref[pl.ds(i * _L, _L)] = jnp.where(m, 1.0, 0.0)

    def chunk(it, carry):
        base = wid * per_w + it * _CHUNK
        pltpu.sync_copy(t_hbm.at[pl.ds(base, _CHUNK)], t_idx)
        pltpu.sync_copy(c_hbm.at[pl.ds(base, _CHUNK)], c_idx)
        pltpu.sync_copy(n_hbm.at[pl.ds(base * K, _CHUNK * K)], n_idx)
        split_idx(t_idx, tq, ht, _CHUNK // _L)
        split_idx(c_idx, cq, hc, _CHUNK // _L)
        split_idx(n_idx, nq, hn, (_CHUNK * K) // _L)
        cps = [
            pltpu.async_copy(iw_hbm.at[tq], v_rows, sem),
            pltpu.async_copy(ow_hbm.at[cq], up_rows, sem),
        ]
        for g in range(ng):
            cps.append(pltpu.async_copy(
                ow_hbm.at[nq.at[pl.ds(g * 128, 128)]],
                un_rows.at[pl.ds(g * 128, 128)], sem))
        for cp in cps:
            cp.wait()

        def splat(h_ref, i):
            grp = (i // _L) * _L
            hvec = h_ref[pl.ds(grp, _L)]
            return hvec.at[jnp.broadcast_to(i - grp, (_L,))].get(
                mode="promise_in_bounds")

        def blend(rows, b, h_s):
            out = []
            for j in range(nd):
                lo = rows[b, pl.ds(j * _L, _L)]
                hi = rows[b, pl.ds(D + j * _L, _L)]
                out.append(lo + h_s * (hi - lo))
            return out

        def elem(b, carry2):
            v = blend(v_rows, b, splat(ht, b))
            u = blend(up_rows, b, splat(hc, b))
            acc = v[0] * u[0]
            for j in range(1, nd):
                acc = acc + v[j] * u[j]
            pos_part[pl.ds(b * _L, _L)] = acc
            for k in range(K):
                r = b * K + k
                w = blend(un_rows, r, splat(hn, r))
                nacc = v[0] * w[0]
                for j in range(1, nd):
                    nacc = nacc + v[j] * w[j]
                neg_part[pl.ds(r * _L, _L)] = nacc
            return carry2

        lax.fori_loop(0, _CHUNK, elem, 0)
        pltpu.sync_copy(pos_part, pos_out.at[pl.ds(base * _L, _CHUNK * _L)])
        pltpu.sync_copy(neg_part,
                        neg_out.at[pl.ds(base * K * _L, _CHUNK * K * _L)])
        return carry

    lax.fori_loop(0, n_chunks, chunk, 0)


@functools.lru_cache(maxsize=None)
def _make_sc(B, K, D, H):
    mesh = plsc.VectorSubcoreMesh(core_axis_name="c", subcore_axis_name="s")
    return pl.kernel(
        functools.partial(_sc_body, K, D, H),
        out_type=[
            jax.ShapeDtypeStruct((B * _L,), jnp.float32),
            jax.ShapeDtypeStruct((B * K * _L,), jnp.float32),
        ],
        mesh=mesh,
        compiler_params=pltpu.CompilerParams(use_tc_tiling_on_sc=False),
        scratch_types=[
            pltpu.VMEM((_CHUNK,), jnp.int32),
            pltpu.VMEM((_CHUNK,), jnp.int32),
            pltpu.VMEM((_CHUNK * K,), jnp.int32),
            pltpu.VMEM((_CHUNK,), jnp.int32),
            pltpu.VMEM((_CHUNK,), jnp.int32),
            pltpu.VMEM((_CHUNK * K,), jnp.int32),
            pltpu.VMEM((_CHUNK,), jnp.float32),
            pltpu.VMEM((_CHUNK,), jnp.float32),
            pltpu.VMEM((_CHUNK * K,), jnp.float32),
            pltpu.VMEM((_CHUNK, 2 * D), jnp.float32),
            pltpu.VMEM((_CHUNK, 2 * D), jnp.float32),
            pltpu.VMEM((_CHUNK * K, 2 * D), jnp.float32),
            pltpu.VMEM((_CHUNK * _L,), jnp.float32),
            pltpu.VMEM((_CHUNK * K * _L,), jnp.float32),
            pltpu.SemaphoreType.DMA,
        ],
    )


def _tr_body(alo_ref, ahi_ref, blo_ref, bhi_ref, ao_ref, bo_ref):
    d = alo_ref.shape[0]
    eye = (lax.broadcasted_iota(jnp.int32, (d, d), 0)
           == lax.broadcasted_iota(jnp.int32, (d, d), 1)).astype(jnp.float32)
    dn = (((0,), (0,)), ((), ()))

    def tr(x):
        return lax.dot_general(x, eye, dn, preferred_element_type=jnp.float32)

    ao_ref[:] = jnp.concatenate([tr(alo_ref[:]), tr(ahi_ref[:])], axis=1)
    bo_ref[:] = jnp.concatenate([tr(blo_ref[:]), tr(bhi_ref[:])], axis=1)


_TBLK = 8192


def _pair_offset(V):
    return (V // (2 * _TBLK)) * _TBLK


def _tc_transpose(iw_t, ow_t):
    D, V = iw_t.shape
    hp_b = _pair_offset(V) // _TBLK
    R = V - hp_b * _TBLK
    grid = (R + _TBLK - 1) // _TBLK
    return pl.pallas_call(
        _tr_body,
        grid=(grid,),
        in_specs=[
            pl.BlockSpec((D, _TBLK), lambda i: (0, i)),
            pl.BlockSpec((D, _TBLK), lambda i, o=hp_b: (0, o + i)),
            pl.BlockSpec((D, _TBLK), lambda i: (0, i)),
            pl.BlockSpec((D, _TBLK), lambda i, o=hp_b: (0, o + i)),
        ],
        out_specs=[
            pl.BlockSpec((_TBLK, 2 * D), lambda i: (i, 0)),
            pl.BlockSpec((_TBLK, 2 * D), lambda i: (i, 0)),
        ],
        out_shape=[
            jax.ShapeDtypeStruct((R, 2 * D), jnp.float32),
            jax.ShapeDtypeStruct((R, 2 * D), jnp.float32),
        ],
    )(iw_t, iw_t, ow_t, ow_t)


def _loss_body(B, pos_ref, neg_ref, out_ref):
    i = pl.program_id(0)

    @pl.when(i == 0)
    def _init():
        out_ref[0, 0] = 0.0

    col = lax.broadcasted_iota(jnp.int32, (128, 128), 0) // _L
    row = lax.broadcasted_iota(jnp.int32, (128, 128), 1)
    sel = jnp.where((col == row) & (row < 128 // _L), 1.0, 0.0)
    valid = lax.broadcasted_iota(jnp.int32, (1, 128), 1) < 128 // _L

    def logsig(x):
        return jnp.minimum(x, 0.0) - jnp.log1p(jnp.exp(-jnp.abs(x)))

    ps = jnp.dot(pos_ref[:], sel, preferred_element_type=jnp.float32)
    ns = jnp.dot(neg_ref[:], sel, preferred_element_type=jnp.float32)
    tot = (jnp.sum(jnp.where(valid, logsig(ps), 0.0))
           + jnp.sum(jnp.where(valid, logsig(-ns), 0.0)))
    out_ref[0, 0] += -tot / B


def _tc_loss(pos2d, neg2d, B):
    n_steps = 8
    pb = pos2d.shape[0] // n_steps
    nb = neg2d.shape[0] // n_steps
    out = pl.pallas_call(
        functools.partial(_loss_body, B),
        grid=(n_steps,),
        in_specs=[
            pl.BlockSpec((pb, 128), lambda i: (i, 0)),
            pl.BlockSpec((nb, 128), lambda i: (i, 0)),
        ],
        out_specs=pl.BlockSpec(memory_space=pltpu.SMEM),
        out_shape=jax.ShapeDtypeStruct((1, 1), jnp.float32),
    )(pos2d, neg2d)
    return out[0, 0]


def kernel(targets, contexts, negatives, input_w, output_w):
    B = targets.shape[0]
    K = negatives.shape[1]
    D = input_w.shape[1]
    neg_flat = negatives.reshape(B * K)
    iw_pk, ow_pk = _tc_transpose(input_w.T, output_w.T)
    pos_part, neg_part = _make_sc(B, K, D, _pair_offset(input_w.shape[0]))(
        targets, contexts, neg_flat, iw_pk, ow_pk)
    pos2d = pos_part.reshape(-1, 128)
    neg2d = neg_part.reshape(-1, 128)
    return _tc_loss(pos2d, neg2d, B)


# speedup vs baseline: 2.4145x; 1.0425x over previous
import functools

import jax
import jax.numpy as jnp
from jax import lax
from jax.experimental import pallas as pl
from jax.experimental.pallas import tpu as pltpu
from jax.experimental.pallas import tpu_sc as plsc

_NC = 2
_NS = 16
_L = 16
_NW = _NC * _NS
_CHUNK = 32
_GS = 64


_NSET = 14


def _sc_body(K, D, H, t_hbm, c_hbm, n_hbm, iw_hbm, ow_hbm, pos_out, neg_out,
             *scr):
    sets = [scr[i * _NSET:(i + 1) * _NSET] for i in range(2)]
    sems = scr[2 * _NSET:]
    B = t_hbm.shape[0]
    per_w = B // _NW
    n_chunks = per_w // _CHUNK
    n_pairs = n_chunks // 2
    nd = D // _L
    ng = (_CHUNK * K) // _GS
    wid = lax.axis_index("s") * _NC + lax.axis_index("c")

    def split_idx(idx_ref, q_ref, h_ref, n16):
        for i in range(n16):
            x = idx_ref[pl.ds(i * _L, _L)]
            m = x >= H
            q_ref[pl.ds(i * _L, _L)] = jnp.where(m, x - H, x)
            h_ref[pl.ds(i * _L, _L)] = jnp.where(m, 1.0, 0.0)

    def gathers(s, sem):
        (t_idx, c_idx, n_idx, tq, cq, nq, ht, hc, hn, v_rows, up_rows,
         un_rows, pos_part, neg_part) = s
        cps = [
            pltpu.make_async_copy(iw_hbm.at[tq], v_rows, sem),
            pltpu.make_async_copy(ow_hbm.at[cq], up_rows, sem),
        ]
        for g in range(ng):
            cps.append(pltpu.make_async_copy(
                ow_hbm.at[nq.at[pl.ds(g * _GS, _GS)]],
                un_rows.at[pl.ds(g * _GS, _GS)], sem))
        return cps

    def fire(it, s, sem):
        (t_idx, c_idx, n_idx, tq, cq, nq, ht, hc, hn, v_rows, up_rows,
         un_rows, pos_part, neg_part) = s
        base = wid * per_w + it * _CHUNK
        pltpu.sync_copy(t_hbm.at[pl.ds(base, _CHUNK)], t_idx)
        pltpu.sync_copy(c_hbm.at[pl.ds(base, _CHUNK)], c_idx)
        pltpu.sync_copy(n_hbm.at[pl.ds(base * K, _CHUNK * K)], n_idx)
        split_idx(t_idx, tq, ht, _CHUNK // _L)
        split_idx(c_idx, cq, hc, _CHUNK // _L)
        split_idx(n_idx, nq, hn, (_CHUNK * K) // _L)
        for cp in gathers(s, sem):
            cp.start()

    def drain(s, sem):
        for cp in gathers(s, sem):
            cp.wait()

    def compute(it, s):
        (t_idx, c_idx, n_idx, tq, cq, nq, ht, hc, hn, v_rows, up_rows,
         un_rows, pos_part, neg_part) = s
        base = wid * per_w + it * _CHUNK

        def splat(h_ref, i):
            grp = (i // _L) * _L
            hvec = h_ref[pl.ds(grp, _L)]
            return hvec.at[jnp.broadcast_to(i - grp, (_L,))].get(
                mode="promise_in_bounds")

        def blend(rows, b, h_s):
            out = []
            for j in range(nd):
                lo = rows[b, pl.ds(j * _L, _L)]
                hi = rows[b, pl.ds(D + j * _L, _L)]
                out.append(lo + h_s * (hi - lo))
            return out

        def elem(b, carry2):
            v = blend(v_rows, b, splat(ht, b))
            u = blend(up_rows, b, splat(hc, b))
            acc = v[0] * u[0]
            for j in range(1, nd):
                acc = acc + v[j] * u[j]
            pos_part[pl.ds(b * _L, _L)] = acc
            for k in range(K):
                r = b * K + k
                w = blend(un_rows, r, splat(hn, r))
                nacc = v[0] * w[0]
                for j in range(1, nd):
                    nacc = nacc + v[j] * w[j]
                neg_part[pl.ds(r * _L, _L)] = nacc
            return carry2

        lax.fori_loop(0, _CHUNK, elem, 0)
        pltpu.sync_copy(pos_part, pos_out.at[pl.ds(base * _L, _CHUNK * _L)])
        pltpu.sync_copy(neg_part,
                        neg_out.at[pl.ds(base * K * _L, _CHUNK * K * _L)])

    fire(0, sets[0], sems[0])

    def pair(j, carry):
        fire(2 * j + 1, sets[1], sems[1])
        drain(sets[0], sems[0])
        compute(2 * j, sets[0])
        fire(lax.rem(2 * j + 2, n_chunks), sets[0], sems[0])
        drain(sets[1], sems[1])
        compute(2 * j + 1, sets[1])
        return carry

    lax.fori_loop(0, n_pairs, pair, 0)
    drain(sets[0], sems[0])


@functools.lru_cache(maxsize=None)
def _make_sc(B, K, D, H):
    mesh = plsc.VectorSubcoreMesh(core_axis_name="c", subcore_axis_name="s")
    one_set = [
        pltpu.VMEM((_CHUNK,), jnp.int32),
        pltpu.VMEM((_CHUNK,), jnp.int32),
        pltpu.VMEM((_CHUNK * K,), jnp.int32),
        pltpu.VMEM((_CHUNK,), jnp.int32),
        pltpu.VMEM((_CHUNK,), jnp.int32),
        pltpu.VMEM((_CHUNK * K,), jnp.int32),
        pltpu.VMEM((_CHUNK,), jnp.float32),
        pltpu.VMEM((_CHUNK,), jnp.float32),
        pltpu.VMEM((_CHUNK * K,), jnp.float32),
        pltpu.VMEM((_CHUNK, 2 * D), jnp.float32),
        pltpu.VMEM((_CHUNK, 2 * D), jnp.float32),
        pltpu.VMEM((_CHUNK * K, 2 * D), jnp.float32),
        pltpu.VMEM((_CHUNK * _L,), jnp.float32),
        pltpu.VMEM((_CHUNK * K * _L,), jnp.float32),
    ]
    return pl.kernel(
        functools.partial(_sc_body, K, D, H),
        out_type=[
            jax.ShapeDtypeStruct((B * _L,), jnp.float32),
            jax.ShapeDtypeStruct((B * K * _L,), jnp.float32),
        ],
        mesh=mesh,
        compiler_params=pltpu.CompilerParams(use_tc_tiling_on_sc=False),
        scratch_types=(one_set + one_set
                       + [pltpu.SemaphoreType.DMA, pltpu.SemaphoreType.DMA]),
    )


def _tr_body(alo_ref, ahi_ref, blo_ref, bhi_ref, ao_ref, bo_ref):
    d = alo_ref.shape[0]
    eye = (lax.broadcasted_iota(jnp.int32, (d, d), 0)
           == lax.broadcasted_iota(jnp.int32, (d, d), 1)).astype(jnp.float32)
    dn = (((0,), (0,)), ((), ()))

    def tr(x):
        return lax.dot_general(x, eye, dn, preferred_element_type=jnp.float32)

    ao_ref[:] = jnp.concatenate([tr(alo_ref[:]), tr(ahi_ref[:])], axis=1)
    bo_ref[:] = jnp.concatenate([tr(blo_ref[:]), tr(bhi_ref[:])], axis=1)


_TBLK = 8192


def _pair_offset(V):
    return (V // (2 * _TBLK)) * _TBLK


def _tc_transpose(iw_t, ow_t):
    D, V = iw_t.shape
    hp_b = _pair_offset(V) // _TBLK
    R = V - hp_b * _TBLK
    grid = (R + _TBLK - 1) // _TBLK
    return pl.pallas_call(
        _tr_body,
        grid=(grid,),
        in_specs=[
            pl.BlockSpec((D, _TBLK), lambda i: (0, i)),
            pl.BlockSpec((D, _TBLK), lambda i, o=hp_b: (0, o + i)),
            pl.BlockSpec((D, _TBLK), lambda i: (0, i)),
            pl.BlockSpec((D, _TBLK), lambda i, o=hp_b: (0, o + i)),
        ],
        out_specs=[
            pl.BlockSpec((_TBLK, 2 * D), lambda i: (i, 0)),
            pl.BlockSpec((_TBLK, 2 * D), lambda i: (i, 0)),
        ],
        out_shape=[
            jax.ShapeDtypeStruct((R, 2 * D), jnp.float32),
            jax.ShapeDtypeStruct((R, 2 * D), jnp.float32),
        ],
    )(iw_t, iw_t, ow_t, ow_t)


def _loss_body(B, pos_ref, neg_ref, out_ref):
    i = pl.program_id(0)

    @pl.when(i == 0)
    def _init():
        out_ref[0, 0] = 0.0

    col = lax.broadcasted_iota(jnp.int32, (128, 128), 0) // _L
    row = lax.broadcasted_iota(jnp.int32, (128, 128), 1)
    sel = jnp.where((col == row) & (row < 128 // _L), 1.0, 0.0)
    valid = lax.broadcasted_iota(jnp.int32, (1, 128), 1) < 128 // _L

    def logsig(x):
        return jnp.minimum(x, 0.0) - jnp.log1p(jnp.exp(-jnp.abs(x)))

    ps = jnp.dot(pos_ref[:], sel, preferred_element_type=jnp.float32)
    ns = jnp.dot(neg_ref[:], sel, preferred_element_type=jnp.float32)
    tot = (jnp.sum(jnp.where(valid, logsig(ps), 0.0))
           + jnp.sum(jnp.where(valid, logsig(-ns), 0.0)))
    out_ref[0, 0] += -tot / B


def _tc_loss(pos2d, neg2d, B):
    n_steps = 8
    pb = pos2d.shape[0] // n_steps
    nb = neg2d.shape[0] // n_steps
    out = pl.pallas_call(
        functools.partial(_loss_body, B),
        grid=(n_steps,),
        in_specs=[
            pl.BlockSpec((pb, 128), lambda i: (i, 0)),
            pl.BlockSpec((nb, 128), lambda i: (i, 0)),
        ],
        out_specs=pl.BlockSpec(memory_space=pltpu.SMEM),
        out_shape=jax.ShapeDtypeStruct((1, 1), jnp.float32),
    )(pos2d, neg2d)
    return out[0, 0]


def kernel(targets, contexts, negatives, input_w, output_w):
    B = targets.shape[0]
    K = negatives.shape[1]
    D = input_w.shape[1]
    neg_flat = negatives.reshape(B * K)
    iw_pk, ow_pk = _tc_transpose(input_w.T, output_w.T)
    pos_part, neg_part = _make_sc(B, K, D, _pair_offset(input_w.shape[0]))(
        targets, contexts, neg_flat, iw_pk, ow_pk)
    pos2d = pos_part.reshape(-1, 128)
    neg2d = neg_part.reshape(-1, 128)
    return _tc_loss(pos2d, neg2d, B)
